# baseline (speedup 1.0000x reference)
import jax
import jax.numpy as jnp
from jax.experimental import pallas as pl
from jax.experimental.pallas import tpu as pltpu


def _affine_body(ids_ref, w_ref, b_ref, x_ref, o_ref):
    i = pl.program_id(0)
    c = pl.program_id(1)
    sid = ids_ref[i]
    num_sat = w_ref.shape[0]
    valid = jnp.logical_and(sid >= 0, sid < num_sat)
    s = jnp.where(valid, sid, 0)
    w = jnp.where(valid, w_ref[s, c], jnp.float32(1.0))
    b = jnp.where(valid, b_ref[s, c], jnp.float32(0.0))
    o_ref[...] = x_ref[...] * w + b


def kernel(x, satellite_ids, weight, bias):
    B, N, C, H, W = x.shape
    S = weight.shape[0]
    xr = x.reshape(B * N, C, H, W)
    ids = satellite_ids.reshape(-1).astype(jnp.int32)
    w2 = weight.reshape(S, C)
    b2 = bias.reshape(S, C)
    grid_spec = pltpu.PrefetchScalarGridSpec(
        num_scalar_prefetch=3,
        grid=(B * N, C),
        in_specs=[
            pl.BlockSpec((1, 1, H, W), lambda i, c, ids, w, b: (i, c, 0, 0)),
        ],
        out_specs=pl.BlockSpec((1, 1, H, W), lambda i, c, ids, w, b: (i, c, 0, 0)),
    )
    out = pl.pallas_call(
        _affine_body,
        grid_spec=grid_spec,
        out_shape=jax.ShapeDtypeStruct((B * N, C, H, W), x.dtype),
        compiler_params=pltpu.CompilerParams(
            dimension_semantics=("arbitrary", "arbitrary"),
        ),
    )(ids, w2, b2, xr)
    return out.reshape(B, N, C, H, W)


# speedup vs baseline: 1.0199x; 1.0199x over previous
import jax
import jax.numpy as jnp
from jax.experimental import pallas as pl
from jax.experimental.pallas import tpu as pltpu


def _affine_body(ids_ref, w_ref, b_ref, x_ref, o_ref):
    i = pl.program_id(0)
    c = pl.program_id(1)
    sid = ids_ref[i]
    num_sat = w_ref.shape[0]
    valid = jnp.logical_and(sid >= 0, sid < num_sat)
    s = jnp.where(valid, sid, 0)
    w = jnp.where(valid, w_ref[s, c], jnp.float32(1.0))
    b = jnp.where(valid, b_ref[s, c], jnp.float32(0.0))
    o_ref[...] = x_ref[...] * w + b


def kernel(x, satellite_ids, weight, bias):
    B, N, C, H, W = x.shape
    S = weight.shape[0]
    xr = x.reshape(B * N, C, H, W)
    ids = satellite_ids.reshape(-1).astype(jnp.int32)
    w2 = weight.reshape(S, C)
    b2 = bias.reshape(S, C)
    grid_spec = pltpu.PrefetchScalarGridSpec(
        num_scalar_prefetch=3,
        grid=(B * N, C),
        in_specs=[
            pl.BlockSpec((1, 1, H, W), lambda i, c, ids, w, b: (i, c, 0, 0)),
        ],
        out_specs=pl.BlockSpec((1, 1, H, W), lambda i, c, ids, w, b: (i, c, 0, 0)),
    )
    out = pl.pallas_call(
        _affine_body,
        grid_spec=grid_spec,
        out_shape=jax.ShapeDtypeStruct((B * N, C, H, W), x.dtype),
        compiler_params=pltpu.CompilerParams(
            dimension_semantics=("parallel", "parallel"),
        ),
    )(ids, w2, b2, xr)
    return out.reshape(B, N, C, H, W)


# speedup vs baseline: 1.4711x; 1.4424x over previous
import jax
import jax.numpy as jnp
from jax.experimental import pallas as pl
from jax.experimental.pallas import tpu as pltpu


def _affine_body(ids_ref, w_ref, b_ref, x_ref, o_ref):
    i = pl.program_id(0)
    sid = ids_ref[i]
    num_sat = w_ref.shape[0]
    C = x_ref.shape[1]
    valid = jnp.logical_and(sid >= 0, sid < num_sat)
    s = jnp.where(valid, sid, 0)
    for c in range(C):
        w = jnp.where(valid, w_ref[s, c], jnp.float32(1.0))
        b = jnp.where(valid, b_ref[s, c], jnp.float32(0.0))
        o_ref[0, c] = x_ref[0, c] * w + b


def kernel(x, satellite_ids, weight, bias):
    B, N, C, H, W = x.shape
    S = weight.shape[0]
    xr = x.reshape(B * N, C, H, W)
    ids = satellite_ids.reshape(-1).astype(jnp.int32)
    w2 = weight.reshape(S, C)
    b2 = bias.reshape(S, C)
    grid_spec = pltpu.PrefetchScalarGridSpec(
        num_scalar_prefetch=3,
        grid=(B * N,),
        in_specs=[
            pl.BlockSpec((1, C, H, W), lambda i, ids, w, b: (i, 0, 0, 0)),
        ],
        out_specs=pl.BlockSpec((1, C, H, W), lambda i, ids, w, b: (i, 0, 0, 0)),
    )
    out = pl.pallas_call(
        _affine_body,
        grid_spec=grid_spec,
        out_shape=jax.ShapeDtypeStruct((B * N, C, H, W), x.dtype),
        compiler_params=pltpu.CompilerParams(
            dimension_semantics=("parallel",),
        ),
    )(ids, w2, b2, xr)
    return out.reshape(B, N, C, H, W)


# speedup vs baseline: 1.5234x; 1.0355x over previous
import jax
import jax.numpy as jnp
from jax.experimental import pallas as pl
from jax.experimental.pallas import tpu as pltpu


_SAMPLES_PER_BLOCK = 2


def _affine_body(ids_ref, w_ref, b_ref, x_ref, o_ref):
    i = pl.program_id(0)
    num_sat = w_ref.shape[0]
    P, C = x_ref.shape[0], x_ref.shape[1]
    for j in range(P):
        sid = ids_ref[i * P + j]
        valid = jnp.logical_and(sid >= 0, sid < num_sat)
        s = jnp.where(valid, sid, 0)
        for c in range(C):
            w = jnp.where(valid, w_ref[s, c], jnp.float32(1.0))
            b = jnp.where(valid, b_ref[s, c], jnp.float32(0.0))
            o_ref[j, c] = x_ref[j, c] * w + b


def kernel(x, satellite_ids, weight, bias):
    B, N, C, H, W = x.shape
    S = weight.shape[0]
    xr = x.reshape(B * N, C, H, W)
    ids = satellite_ids.reshape(-1).astype(jnp.int32)
    w2 = weight.reshape(S, C)
    b2 = bias.reshape(S, C)
    P = _SAMPLES_PER_BLOCK
    grid_spec = pltpu.PrefetchScalarGridSpec(
        num_scalar_prefetch=3,
        grid=(B * N // P,),
        in_specs=[
            pl.BlockSpec((P, C, H, W), lambda i, ids, w, b: (i, 0, 0, 0)),
        ],
        out_specs=pl.BlockSpec((P, C, H, W), lambda i, ids, w, b: (i, 0, 0, 0)),
    )
    out = pl.pallas_call(
        _affine_body,
        grid_spec=grid_spec,
        out_shape=jax.ShapeDtypeStruct((B * N, C, H, W), x.dtype),
        compiler_params=pltpu.CompilerParams(
            dimension_semantics=("parallel",),
        ),
    )(ids, w2, b2, xr)
    return out.reshape(B, N, C, H, W)
